# baseline (speedup 1.0000x reference)
import jax
import jax.numpy as jnp
from jax import lax
from jax.experimental import pallas as pl
from jax.experimental.pallas import tpu as pltpu
from jax.experimental.pallas import tpu_sc as plsc

VOCAB = 100000
NGRAM_VOCAB = 1000000
DIM = 64
BATCH = 4096
NGRAMS = 20
PW = 2 * DIM

NC, NS, L = 2, 16, 16
NW = NC * NS
BW = BATCH // NW
DV = DIM // L
NBUF = 2


def _body(we, ne, cwi, cnt, xwi, xnt, out,
          tnv, tgt, widx, wtgt, rows, vacc, cflat, xflat, outv,
          shacc, sem_w0, sem_w1, gs0, gs1, ss0, ss1):
    cid = lax.axis_index("c")
    sid = lax.axis_index("s")
    wid = sid * NC + cid
    base = wid * BW
    iota = jax.lax.iota(jnp.int32, L)
    gsems = (gs0, gs1)
    ssems = (ss0, ss1)
    wsems = (sem_w0, sem_w1)
    abase = [sid * 2 * BW, sid * 2 * BW]

    for k, (wsrc, nsrc) in enumerate(((cwi, cnt), (xwi, xnt))):
        pltpu.sync_copy(wsrc.at[pl.ds(base, BW)], widx.at[k])
        pltpu.sync_copy(nsrc.at[:, pl.ds(base, BW)], tnv.at[k])

    for k in range(2):
        for i in range(BW // L):
            v = widx[k, pl.ds(i * L, L)]
            wtgt[k, pl.ds(i * L, L)] = (
                abase[k] + 2 * (i * L + iota) + lax.bitwise_and(v, 1))
            widx[k, pl.ds(i * L, L)] = lax.shift_right_logical(v, 1)

        @pl.loop(0, NGRAMS)
        def _(g):
            for i in range(BW // L):
                v = tnv[k, g, pl.ds(i * L, L)]
                tgt[k, g, pl.ds(i * L, L)] = (
                    abase[k] + 2 * (i * L + iota) + lax.bitwise_and(v, 1))
                tnv[k, g, pl.ds(i * L, L)] = lax.shift_right_logical(v, 1)

    for k, flat in ((0, cflat), (1, xflat)):
        @pl.loop(0, BW)
        def _(r):
            for c in range(PW // L):
                rows[0, r, pl.ds(c * L, L)] = jnp.zeros((L,), jnp.float32)

        for q in range(2):
            pltpu.sync_copy(rows.at[0],
                            shacc.at[pl.ds(sid * 2 * BW + q * BW, BW), :])

        pltpu.async_copy(we.at[widx.at[k]], rows.at[0], wsems[k]).wait()
        pltpu.sync_copy(rows.at[0], shacc.at[wtgt.at[k]], add=True)

        gather_cp = [None] * NBUF
        sct_cp = [None] * NBUF

        def fire_gather(g, k=k):
            j = g % NBUF
            gather_cp[j] = pltpu.async_copy(
                ne.at[tnv.at[k, g]], rows.at[j], gsems[j])

        for g in range(NBUF):
            fire_gather(g)

        for g in range(NGRAMS):
            j = g % NBUF
            gather_cp[j].wait()
            sct_cp[j] = pltpu.async_copy(
                rows.at[j], shacc.at[tgt.at[k, g]], ssems[j], add=True)
            if g + NBUF < NGRAMS:
                sct_cp[j].wait()
                sct_cp[j] = None
                fire_gather(g + NBUF)

        for j in range(NBUF):
            if sct_cp[j] is not None:
                sct_cp[j].wait()

        for q in range(2):
            pltpu.sync_copy(shacc.at[pl.ds(abase[k] + q * BW, BW), :], vacc)

            @pl.loop(0, BW // 2)
            def _(r):
                for c in range(DV):
                    flat[pl.ds((q * (BW // 2) + r) * DIM + c * L, L)] = (
                        vacc[2 * r, pl.ds(c * L, L)]
                        + vacc[2 * r + 1, pl.ds(DIM + c * L, L)])

    for b0 in range(BW // L):
        ridx = (b0 * L + iota) * DIM

        @pl.loop(0, DIM, init_carry=jnp.zeros((L,), jnp.float32))
        def s(d, s):
            c = plsc.load_gather(cflat, [ridx + d])
            x = plsc.load_gather(xflat, [ridx + d])
            return s + c * x

        outv[pl.ds(b0 * L, L)] = s

    pltpu.sync_copy(outv, out.at[pl.ds(base, BW)])


@jax.jit
def _run(we, ne, cwi, cnt, xwi, xnt):
    mesh = plsc.VectorSubcoreMesh(core_axis_name="c", subcore_axis_name="s",
                                  num_cores=NC, num_subcores=NS)
    f = pl.kernel(
        _body,
        out_type=jax.ShapeDtypeStruct((BATCH,), jnp.float32),
        mesh=mesh,
        compiler_params=pltpu.CompilerParams(
            needs_layout_passes=False, use_tc_tiling_on_sc=False),
        scratch_types=[
            pltpu.VMEM((2, NGRAMS, BW), jnp.int32),
            pltpu.VMEM((2, NGRAMS, BW), jnp.int32),
            pltpu.VMEM((2, BW), jnp.int32),
            pltpu.VMEM((2, BW), jnp.int32),
            pltpu.VMEM((NBUF, BW, PW), jnp.float32),
            pltpu.VMEM((BW, PW), jnp.float32),
            pltpu.VMEM((BW * DIM,), jnp.float32),
            pltpu.VMEM((BW * DIM,), jnp.float32),
            pltpu.VMEM((BW,), jnp.float32),
            pltpu.VMEM_SHARED((NS * 2 * BW, PW), jnp.float32),
            pltpu.SemaphoreType.DMA,
            pltpu.SemaphoreType.DMA,
            pltpu.SemaphoreType.DMA,
            pltpu.SemaphoreType.DMA,
            pltpu.SemaphoreType.DMA,
            pltpu.SemaphoreType.DMA,
        ],
    )
    return f(we, ne, cwi, cnt, xwi, xnt)


def kernel(word_embeddings, ngram_embeddings, center_word_idx,
           center_ngram_idxs, context_word_idx, context_ngram_idxs):
    return _run(
        word_embeddings.reshape(VOCAB // 2, PW),
        ngram_embeddings.reshape(NGRAM_VOCAB // 2, PW),
        center_word_idx.astype(jnp.int32),
        center_ngram_idxs.astype(jnp.int32).T,
        context_word_idx.astype(jnp.int32),
        context_ngram_idxs.astype(jnp.int32).T)
